# baseline (speedup 1.0000x reference)
import functools

import jax
import jax.numpy as jnp
from jax import lax
from jax.experimental import pallas as pl
from jax.experimental.pallas import tpu as pltpu
from jax.experimental.pallas import tpu_sc as plsc

D = 16
S = 50
NC, NS = 2, 16
NW = NC * NS
R_TOTAL = 16384
R_PER_W = R_TOTAL // NW
R_CHUNK = 32
CHUNK = R_CHUNK * S
B_PER_W = R_PER_W * S
N_CHUNKS = R_PER_W // R_CHUNK
L = 16
PACK_ROWS = CHUNK * D // 128
OUT_ROWS = R_TOTAL * S * D // 128


def _emb_body(idx_hbm, table_hbm, out_hbm, idx2d, idx_v,
              rows0, rows1, pack,
              gsem0, gsem1, wsem0, wsem1):
    wid = lax.axis_index("s") * NC + lax.axis_index("c")
    row0 = wid * R_PER_W
    rows = (rows0, rows1)
    gsems = (gsem0, gsem1)
    wsems = (wsem0, wsem1)
    pltpu.sync_copy(idx_hbm.at[pl.ds(row0, R_PER_W), :], idx2d)

    def repack_idx(it, carry):
        jv, cv = carry
        v = plsc.load_gather(idx2d, [jv, cv])
        idx_v[pl.ds(it * L, L)] = v
        cv = cv + L
        w = (cv >= S).astype(jnp.int32)
        return (jv + w, cv - S * w)

    lax.fori_loop(0, B_PER_W // L, repack_idx,
                  (jnp.zeros((L,), jnp.int32), lax.iota(jnp.int32, L)))

    def fire_gather(i):
        return pltpu.async_copy(
            table_hbm.at[idx_v.at[pl.ds(i * CHUNK, CHUNK)]],
            rows[i % 2], gsems[i % 2])

    def repack_rows(i):
        src = rows[i % 2]
        dst = pack

        def body(r, _):
            v = src[r, :]
            dst[lax.shift_right_logical(r, 3),
                pl.ds((r % 8) * D, D)] = v
            return 0

        lax.fori_loop(0, CHUNK, body, 0, unroll=8)

    gath = [None] * N_CHUNKS
    wb = [None] * N_CHUNKS
    gath[0] = fire_gather(0)
    for i in range(N_CHUNKS):
        if i + 1 < N_CHUNKS:
            gath[i + 1] = fire_gather(i + 1)
        gath[i].wait()
        if i >= 1:
            wb[i - 1].wait()
        repack_rows(i)
        wb[i] = pltpu.async_copy(
            pack,
            out_hbm.at[pl.ds(wid * (B_PER_W * D // 128) + i * PACK_ROWS,
                             PACK_ROWS), :],
            wsems[i % 2])
    wb[N_CHUNKS - 1].wait()


@jax.jit
def _embed(idx, W):
    mesh = plsc.VectorSubcoreMesh(core_axis_name="c", subcore_axis_name="s")
    fn = functools.partial(
        pl.kernel,
        mesh=mesh,
        out_type=jax.ShapeDtypeStruct((OUT_ROWS, 128), jnp.float32),
        scratch_types=[
            pltpu.VMEM((R_PER_W, S), jnp.int32),
            pltpu.VMEM((B_PER_W,), jnp.int32),
            pltpu.VMEM((CHUNK, D), jnp.float32),
            pltpu.VMEM((CHUNK, D), jnp.float32),
            pltpu.VMEM((PACK_ROWS, 128), jnp.float32),
            pltpu.SemaphoreType.DMA,
            pltpu.SemaphoreType.DMA,
            pltpu.SemaphoreType.DMA,
            pltpu.SemaphoreType.DMA,
        ],
        compiler_params=pltpu.CompilerParams(
            use_tc_tiling_on_sc=False, needs_layout_passes=False),
    )(_emb_body)
    return fn(idx, W)


def kernel(inputs, W):
    out2d = _embed(inputs, W)
    return jnp.minimum(out2d.reshape(R_TOTAL, S, D), 1.0)


# speedup vs baseline: 1.1088x; 1.1088x over previous
import functools

import jax
import jax.numpy as jnp
from jax import lax
from jax.experimental import pallas as pl
from jax.experimental.pallas import tpu as pltpu
from jax.experimental.pallas import tpu_sc as plsc

D = 16
S = 50
NC, NS = 2, 16
NW = NC * NS
R_TOTAL = 16384
R_PER_W = R_TOTAL // NW
R_CHUNK = 64
CHUNK = R_CHUNK * S
B_PER_W = R_PER_W * S
N_CHUNKS = R_PER_W // R_CHUNK


def _emb_body(idx_hbm, table_hbm, out_hbm, idx_v, rows0, rows1,
              gsem0, gsem1, wsem0, wsem1):
    wid = lax.axis_index("s") * NC + lax.axis_index("c")
    base = wid * B_PER_W
    row0 = wid * R_PER_W
    rows = (rows0, rows1)
    gsems = (gsem0, gsem1)
    wsems = (wsem0, wsem1)
    pltpu.sync_copy(idx_hbm.at[pl.ds(base, B_PER_W)], idx_v)
    gath = [None] * N_CHUNKS
    wb = [None] * N_CHUNKS
    gath[0] = pltpu.async_copy(
        table_hbm.at[idx_v.at[pl.ds(0, CHUNK)]], rows[0], gsems[0])
    for i in range(N_CHUNKS):
        if i + 1 < N_CHUNKS:
            if i >= 1:
                for h in wb[i - 1]:
                    h.wait()
            gath[i + 1] = pltpu.async_copy(
                table_hbm.at[idx_v.at[pl.ds((i + 1) * CHUNK, CHUNK)]],
                rows[(i + 1) % 2], gsems[(i + 1) % 2])
        gath[i].wait()
        wb[i] = [
            pltpu.async_copy(
                rows[i % 2].at[pl.ds(j * S, S), :],
                out_hbm.at[row0 + i * R_CHUNK + j, :, :],
                wsems[i % 2])
            for j in range(R_CHUNK)
        ]
    for h in wb[N_CHUNKS - 2]:
        h.wait()
    for h in wb[N_CHUNKS - 1]:
        h.wait()


@jax.jit
def _embed(idx_flat, W):
    mesh = plsc.VectorSubcoreMesh(core_axis_name="c", subcore_axis_name="s")
    fn = functools.partial(
        pl.kernel,
        mesh=mesh,
        out_type=jax.ShapeDtypeStruct((R_TOTAL, S, D), jnp.float32),
        scratch_types=[
            pltpu.VMEM((B_PER_W,), jnp.int32),
            pltpu.VMEM((CHUNK, D), jnp.float32),
            pltpu.VMEM((CHUNK, D), jnp.float32),
            pltpu.SemaphoreType.DMA,
            pltpu.SemaphoreType.DMA,
            pltpu.SemaphoreType.DMA,
            pltpu.SemaphoreType.DMA,
        ],
        compiler_params=pltpu.CompilerParams(use_tc_tiling_on_sc=False),
    )(_emb_body)
    return fn(idx_flat, W)


def kernel(inputs, W):
    return _embed(inputs.reshape(-1), W)


# speedup vs baseline: 1.1157x; 1.0062x over previous
import functools

import jax
import jax.numpy as jnp
from jax import lax
from jax.experimental import pallas as pl
from jax.experimental.pallas import tpu as pltpu
from jax.experimental.pallas import tpu_sc as plsc

D = 16
S = 50
NC, NS = 2, 16
NW = NC * NS
R_TOTAL = 16384
R_PER_W = R_TOTAL // NW
R_CHUNK = 64
CHUNK = R_CHUNK * S
B_PER_W = R_PER_W * S
N_CHUNKS = R_PER_W // R_CHUNK


def _emb_body(idx_hbm, table_hbm, out_hbm, idx_v, rows0, rows1,
              gsem0, gsem1, wsem0, wsem1):
    wid = lax.axis_index("s") * NC + lax.axis_index("c")
    base = wid * B_PER_W
    row0 = wid * R_PER_W
    rows = (rows0, rows1)
    gsems = (gsem0, gsem1)
    wsems = (wsem0, wsem1)
    pltpu.sync_copy(idx_hbm.at[pl.ds(base, B_PER_W)], idx_v)
    def fire_writebacks(i):
        def body(j, _):
            pltpu.async_copy(
                rows[i % 2].at[pl.ds(j * S, S), :],
                out_hbm.at[row0 + i * R_CHUNK + j, :, :],
                wsems[i % 2])
            return 0

        lax.fori_loop(0, R_CHUNK, body, 0)

    def drain_writebacks(i):
        pltpu.make_async_copy(
            table_hbm.at[pl.ds(0, CHUNK)], rows[i % 2],
            wsems[i % 2]).wait()

    gath = [None] * N_CHUNKS
    gath[0] = pltpu.async_copy(
        table_hbm.at[idx_v.at[pl.ds(0, CHUNK)]], rows[0], gsems[0])
    for i in range(N_CHUNKS):
        if i + 1 < N_CHUNKS:
            if i >= 1:
                drain_writebacks(i - 1)
            gath[i + 1] = pltpu.async_copy(
                table_hbm.at[idx_v.at[pl.ds((i + 1) * CHUNK, CHUNK)]],
                rows[(i + 1) % 2], gsems[(i + 1) % 2])
        gath[i].wait()
        fire_writebacks(i)
    drain_writebacks(N_CHUNKS - 2)
    drain_writebacks(N_CHUNKS - 1)


@jax.jit
def _embed(idx_flat, W):
    mesh = plsc.VectorSubcoreMesh(core_axis_name="c", subcore_axis_name="s")
    fn = functools.partial(
        pl.kernel,
        mesh=mesh,
        out_type=jax.ShapeDtypeStruct((R_TOTAL, S, D), jnp.float32),
        scratch_types=[
            pltpu.VMEM((B_PER_W,), jnp.int32),
            pltpu.VMEM((CHUNK, D), jnp.float32),
            pltpu.VMEM((CHUNK, D), jnp.float32),
            pltpu.SemaphoreType.DMA,
            pltpu.SemaphoreType.DMA,
            pltpu.SemaphoreType.DMA,
            pltpu.SemaphoreType.DMA,
        ],
        compiler_params=pltpu.CompilerParams(use_tc_tiling_on_sc=False),
    )(_emb_body)
    return fn(idx_flat, W)


def kernel(inputs, W):
    return _embed(inputs.reshape(-1), W)
